# Initial kernel scaffold; baseline (speedup 1.0000x reference)
import jax
import jax.numpy as jnp
from jax.experimental import pallas as pl


def kernel(sample, entity_embedding, relation_embedding):
    raise NotImplementedError("write your pallas kernel here")


# speedup vs baseline: 1.3266x; 1.3266x over previous
import functools

import jax
import jax.numpy as jnp
from jax import lax
from jax.experimental import pallas as pl
from jax.experimental.pallas import tpu as pltpu
from jax.experimental.pallas import tpu_sc as plsc

NC = 2
NS = 16
L = 16
NW = NC * NS
B = 4096
D = 128
BPW = B // NW
G = BPW // L
GAMMA = 12.0


def _sc_body(idx_h, idx_r, idx_t, ent, rel, out,
             idxh_v, idxr_v, idxt_v, h_v, r_v, t_v, score_v, sem):
    wid = lax.axis_index("s") * NC + lax.axis_index("c")
    base = wid * BPW

    pltpu.sync_copy(idx_h.at[pl.ds(base, BPW)], idxh_v)
    pltpu.sync_copy(idx_r.at[pl.ds(base, BPW)], idxr_v)
    pltpu.sync_copy(idx_t.at[pl.ds(base, BPW)], idxt_v)

    ch = pltpu.async_copy(ent.at[idxh_v], h_v, sem)
    cr = pltpu.async_copy(rel.at[idxr_v], r_v, sem)
    ct = pltpu.async_copy(ent.at[idxt_v], t_v, sem)
    ch.wait()
    cr.wait()
    ct.wait()

    lane = lax.iota(jnp.int32, L)
    for g in range(G):

        def body(j, score_vec):
            i = g * L + j
            acc = jnp.zeros((L,), jnp.float32)
            for c in range(D // L):
                h = h_v[i, pl.ds(c * L, L)]
                r = r_v[i, pl.ds(c * L, L)]
                t = t_v[i, pl.ds(c * L, L)]
                acc = acc + jnp.abs(h + r - t)
            s = jnp.sum(acc)
            return jnp.where(lane == j, s, score_vec)

        sv = lax.fori_loop(0, L, body, jnp.zeros((L,), jnp.float32))
        score_v[pl.ds(g * L, L)] = GAMMA - sv

    pltpu.sync_copy(score_v, out.at[pl.ds(base, BPW)])


@jax.jit
def kernel(sample, entity_embedding, relation_embedding):
    idx = sample.astype(jnp.int32)
    idx_h = idx[:, 0]
    idx_r = idx[:, 1]
    idx_t = idx[:, 2]

    mesh = plsc.VectorSubcoreMesh(core_axis_name="c", subcore_axis_name="s",
                                  num_cores=NC, num_subcores=NS)
    run = pl.kernel(
        _sc_body,
        out_type=jax.ShapeDtypeStruct((B,), jnp.float32),
        mesh=mesh,
        compiler_params=pltpu.CompilerParams(needs_layout_passes=False),
        scratch_types=[
            pltpu.VMEM((BPW,), jnp.int32),
            pltpu.VMEM((BPW,), jnp.int32),
            pltpu.VMEM((BPW,), jnp.int32),
            pltpu.VMEM((BPW, D), jnp.float32),
            pltpu.VMEM((BPW, D), jnp.float32),
            pltpu.VMEM((BPW, D), jnp.float32),
            pltpu.VMEM((BPW,), jnp.float32),
            pltpu.SemaphoreType.DMA,
        ],
    )
    score = run(idx_h, idx_r, idx_t, entity_embedding, relation_embedding)
    return score.reshape(B, 1)
